# baseline (speedup 1.0000x reference)
import functools

import jax
import jax.numpy as jnp
from jax import lax
from jax.experimental import pallas as pl
from jax.experimental.pallas import tpu as pltpu
from jax.experimental.pallas import tpu_sc as plsc

N = 10000
E = 320000
D = 128

_NC = 2
_NS = 16
_NW = _NC * _NS
_B = 128
_BPW = 80
_HB = 40
_EPAD = _NW * _BPW * _B
_ZROWS = 128
_RHI = 640
_RLO = N - (_NS - 1) * _RHI


def _mm_body(h_ref, m_ref, wh_ref, wm_ref, norm_ref, h2_ref, m2_ref):
    h2 = jnp.dot(h_ref[...], wh_ref[...], preferred_element_type=jnp.float32)
    h2_ref[pl.ds(0, N), :] = h2 * norm_ref[...]
    h2_ref[pl.ds(N, _ZROWS), :] = jnp.zeros((_ZROWS, D), jnp.float32)
    m2_ref[...] = jnp.dot(m_ref[...], wm_ref[...], preferred_element_type=jnp.float32)


_mm = pl.pallas_call(
    _mm_body,
    out_shape=(
        jax.ShapeDtypeStruct((N + _ZROWS, D), jnp.float32),
        jax.ShapeDtypeStruct((N, D), jnp.float32),
    ),
)


def _final_body(agg_ref, m2_ref, norm_ref, bh_ref, bm_ref, out_ref):
    s = (agg_ref[0] + agg_ref[1]) * norm_ref[...]
    s = s + bh_ref[...] + m2_ref[...] + bm_ref[...]
    out_ref[...] = jnp.maximum(s, 0.0)


_final = pl.pallas_call(
    _final_body,
    out_shape=jax.ShapeDtypeStruct((N, D), jnp.float32),
)


_mesh = plsc.VectorSubcoreMesh(core_axis_name="c", subcore_axis_name="s")


@functools.partial(
    pl.kernel,
    out_type=jax.ShapeDtypeStruct((_NC, N, D), jnp.float32),
    mesh=_mesh,
    scratch_types=[
        pltpu.VMEM((_HB, _B), jnp.int32),
        pltpu.VMEM((_HB, _B), jnp.int32),
        [pltpu.VMEM((_B, D), jnp.float32) for _ in range(2)],
        pltpu.VMEM_SHARED((N, D), jnp.float32),
        pltpu.SemaphoreType.DMA,
        pltpu.SemaphoreType.DMA,
    ],
)
def _sc_agg(h2_hbm, src_hbm, dst_hbm, zeros_hbm, out_hbm,
            src_v, dst_v, rows, acc_sh, semg, sems):
    cid = lax.axis_index("c")
    sid = lax.axis_index("s")
    w = cid * _NS + sid

    @pl.when(sid < _NS - 1)
    def _():
        pltpu.sync_copy(zeros_hbm, acc_sh.at[pl.ds(sid * _RHI, _RHI)])

    @pl.when(sid == _NS - 1)
    def _():
        pltpu.sync_copy(zeros_hbm.at[pl.ds(0, _RLO)],
                        acc_sh.at[pl.ds(sid * _RHI, _RLO)])

    plsc.subcore_barrier()

    base = w * _BPW

    def _gather(p, b):
        pltpu.async_copy(h2_hbm.at[src_v.at[p]], rows[b], semg)

    def _wait_gather(b):
        pltpu.make_async_copy(h2_hbm.at[src_v.at[0]], rows[b], semg).wait()

    def _scatter(p, b):
        pltpu.async_copy(rows[b], acc_sh.at[dst_v.at[p]], sems, add=True)

    def _wait_scatter(b):
        pltpu.make_async_copy(rows[b], acc_sh.at[dst_v.at[0]], sems).wait()

    for h in range(_BPW // _HB):
        off = base + h * _HB
        pltpu.sync_copy(src_hbm.at[pl.ds(off, _HB)], src_v)
        pltpu.sync_copy(dst_hbm.at[pl.ds(off, _HB)], dst_v)
        _gather(0, 0)
        _gather(1, 1)
        _wait_gather(0)
        _scatter(0, 0)

        def _pstep(p, b, fire_next=True):
            _wait_scatter(1 - b)
            if fire_next:
                _gather(p + 1, 1 - b)
            _wait_gather(b)
            _scatter(p, b)

        _pstep(1, 1)

        def body(t, carry):
            for k in range(2):
                _pstep(2 * t + 2 + k, k)
            return carry

        lax.fori_loop(0, (_HB - 4) // 2, body, 0)

        _pstep(_HB - 2, 0)
        _pstep(_HB - 1, 1, fire_next=False)
        _wait_scatter(1)

    plsc.subcore_barrier()

    @pl.when(sid < _NS - 1)
    def _():
        pltpu.sync_copy(acc_sh.at[pl.ds(sid * _RHI, _RHI)],
                        out_hbm.at[cid, pl.ds(sid * _RHI, _RHI)])

    @pl.when(sid == _NS - 1)
    def _():
        pltpu.sync_copy(acc_sh.at[pl.ds(sid * _RHI, _RLO)],
                        out_hbm.at[cid, pl.ds(sid * _RHI, _RLO)])


def kernel(h, m, wh, wm, bh, bm, norm, edge_index):
    h2, m2 = _mm(h, m, wh, wm, norm)
    npad = _EPAD - E
    pad_iota = jnp.arange(npad, dtype=jnp.int32)
    src = jnp.concatenate([edge_index[0], N + pad_iota % _ZROWS])
    dst = jnp.concatenate([edge_index[1], pad_iota % N])
    src2d = src.reshape(_EPAD // _B, _B)
    dst2d = dst.reshape(_EPAD // _B, _B)
    zeros = jnp.zeros((_RHI, D), dtype=jnp.float32)
    agg = _sc_agg(h2, src2d, dst2d, zeros)
    return _final(agg, m2, norm, bh.reshape(1, D), bm.reshape(1, D))


# speedup vs baseline: 1.0186x; 1.0186x over previous
import functools

import jax
import jax.numpy as jnp
from jax import lax
from jax.experimental import pallas as pl
from jax.experimental.pallas import tpu as pltpu
from jax.experimental.pallas import tpu_sc as plsc

N = 10000
E = 320000
D = 128

_NC = 2
_NS = 16
_NW = _NC * _NS
_B = 128
_BPW = 80
_HB = 40
_EPAD = _NW * _BPW * _B
_ZROWS = 128
_RHI = 640
_RLO = N - (_NS - 1) * _RHI


def _mm_body(h_ref, wh_ref, norm_ref, h2_ref):
    h2 = jnp.dot(h_ref[...], wh_ref[...], preferred_element_type=jnp.float32)
    h2_ref[pl.ds(0, N), :] = h2 * norm_ref[...]
    h2_ref[pl.ds(N, _ZROWS), :] = jnp.zeros((_ZROWS, D), jnp.float32)


_mm = pl.pallas_call(
    _mm_body,
    out_shape=jax.ShapeDtypeStruct((N + _ZROWS, D), jnp.float32),
)


def _final_body(agg_ref, m_ref, wm_ref, norm_ref, bh_ref, bm_ref, out_ref):
    m2 = jnp.dot(m_ref[...], wm_ref[...], preferred_element_type=jnp.float32)
    s = (agg_ref[0] + agg_ref[1]) * norm_ref[...]
    s = s + bh_ref[...] + m2 + bm_ref[...]
    out_ref[...] = jnp.maximum(s, 0.0)


_final = pl.pallas_call(
    _final_body,
    out_shape=jax.ShapeDtypeStruct((N, D), jnp.float32),
)


_mesh = plsc.VectorSubcoreMesh(core_axis_name="c", subcore_axis_name="s")


@functools.partial(
    pl.kernel,
    out_type=jax.ShapeDtypeStruct((_NC, N, D), jnp.float32),
    mesh=_mesh,
    scratch_types=[
        pltpu.VMEM((_HB, _B), jnp.int32),
        pltpu.VMEM((_HB, _B), jnp.int32),
        [pltpu.VMEM((_B, D), jnp.float32) for _ in range(2)],
        pltpu.VMEM_SHARED((N, D), jnp.float32),
        pltpu.SemaphoreType.DMA,
        pltpu.SemaphoreType.DMA,
    ],
)
def _sc_agg(h2_hbm, src_hbm, dst_hbm, zeros_hbm, out_hbm,
            src_v, dst_v, rows, acc_sh, semg, sems):
    cid = lax.axis_index("c")
    sid = lax.axis_index("s")
    w = cid * _NS + sid

    @pl.when(sid < _NS - 1)
    def _():
        pltpu.sync_copy(zeros_hbm, acc_sh.at[pl.ds(sid * _RHI, _RHI)])

    @pl.when(sid == _NS - 1)
    def _():
        pltpu.sync_copy(zeros_hbm.at[pl.ds(0, _RLO)],
                        acc_sh.at[pl.ds(sid * _RHI, _RLO)])

    plsc.subcore_barrier()

    base = w * _BPW

    def _gather(p, b):
        pltpu.async_copy(h2_hbm.at[src_v.at[p]], rows[b], semg)

    def _wait_gather(b):
        pltpu.make_async_copy(h2_hbm.at[src_v.at[0]], rows[b], semg).wait()

    def _scatter(p, b):
        pltpu.async_copy(rows[b], acc_sh.at[dst_v.at[p]], sems, add=True)

    def _wait_scatter(b):
        pltpu.make_async_copy(rows[b], acc_sh.at[dst_v.at[0]], sems).wait()

    for h in range(_BPW // _HB):
        off = base + h * _HB
        pltpu.sync_copy(src_hbm.at[pl.ds(off, _HB)], src_v)
        pltpu.sync_copy(dst_hbm.at[pl.ds(off, _HB)], dst_v)
        _gather(0, 0)
        _gather(1, 1)
        _wait_gather(0)
        _scatter(0, 0)

        def _pstep(p, b, fire_next=True):
            _wait_scatter(1 - b)
            if fire_next:
                _gather(p + 1, 1 - b)
            _wait_gather(b)
            _scatter(p, b)

        _pstep(1, 1)

        def body(t, carry):
            for k in range(2):
                _pstep(2 * t + 2 + k, k)
            return carry

        lax.fori_loop(0, (_HB - 4) // 2, body, 0)

        _pstep(_HB - 2, 0)
        _pstep(_HB - 1, 1, fire_next=False)
        _wait_scatter(1)

    plsc.subcore_barrier()

    @pl.when(sid < _NS - 1)
    def _():
        pltpu.sync_copy(acc_sh.at[pl.ds(sid * _RHI, _RHI)],
                        out_hbm.at[cid, pl.ds(sid * _RHI, _RHI)])

    @pl.when(sid == _NS - 1)
    def _():
        pltpu.sync_copy(acc_sh.at[pl.ds(sid * _RHI, _RLO)],
                        out_hbm.at[cid, pl.ds(sid * _RHI, _RLO)])


def kernel(h, m, wh, wm, bh, bm, norm, edge_index):
    h2 = _mm(h, wh, norm)
    npad = _EPAD - E
    pad_iota = jnp.arange(npad, dtype=jnp.int32)
    src = jnp.concatenate([edge_index[0], N + pad_iota % _ZROWS])
    dst = jnp.concatenate([edge_index[1], pad_iota % N])
    src2d = src.reshape(_EPAD // _B, _B)
    dst2d = dst.reshape(_EPAD // _B, _B)
    zeros = jnp.zeros((_RHI, D), dtype=jnp.float32)
    agg = _sc_agg(h2, src2d, dst2d, zeros)
    return _final(agg, m, wm, norm, bh.reshape(1, D), bm.reshape(1, D))
